# Initial kernel scaffold; baseline (speedup 1.0000x reference)
import jax
import jax.numpy as jnp
from jax.experimental import pallas as pl


def kernel(x, edge_index, nu, W1, b1, W2, b2, W3, b3):
    raise NotImplementedError("write your pallas kernel here")


# speedup vs baseline: 43.2551x; 43.2551x over previous
import functools

import jax
import jax.numpy as jnp
from jax import lax
from jax.experimental import pallas as pl
from jax.experimental.pallas import tpu as pltpu
from jax.experimental.pallas import tpu_sc as plsc

_NC = 2
_NS = 16
_NW = _NC * _NS
_RW = 128
_K = 8


def _edge_pass_gather(n_pad, n_rows, width):
    base = n_rows // _NW
    nwin = base // _K
    assert n_rows == base * _NW and base % _K == 0 and n_pad % (_NS * 8) == 0
    zr = n_pad // _NS
    mesh = plsc.VectorSubcoreMesh(core_axis_name="c", subcore_axis_name="s")

    @functools.partial(
        pl.kernel,
        out_type=jax.ShapeDtypeStruct((_NC, n_pad, width), jnp.float32),
        mesh=mesh,
        scratch_types=[
            pltpu.VMEM_SHARED((n_pad, width), jnp.float32),
            pltpu.VMEM((_K, _RW), jnp.int32),
            pltpu.VMEM((_K, _RW), jnp.int32),
            pltpu.VMEM((_K, _RW, width), jnp.float32),
            pltpu.SemaphoreType.DMA,
            pltpu.SemaphoreType.DMA,
        ],
        compiler_params=pltpu.CompilerParams(use_tc_tiling_on_sc=False),
    )
    def k(src_hbm, dst_hbm, tab_hbm, z_hbm, out_hbm, acc_sp, idx_s, idx_d,
          gbuf, gsem, ssem):
        c = lax.axis_index("c")
        s = lax.axis_index("s")
        wid = s * _NC + c
        pltpu.sync_copy(z_hbm.at[pl.ds(s * zr, zr)], acc_sp.at[pl.ds(s * zr, zr)])
        plsc.subcore_barrier()
        start = wid * base

        def win(w, carry):
            r0 = start + w * _K
            pltpu.sync_copy(src_hbm.at[pl.ds(r0, _K)], idx_s)
            pltpu.sync_copy(dst_hbm.at[pl.ds(r0, _K)], idx_d)
            descs = [pltpu.async_copy(tab_hbm.at[idx_s.at[j]], gbuf.at[j], gsem)
                     for j in range(_K)]
            for d in descs:
                d.wait()
            descs = [pltpu.async_copy(gbuf.at[j], acc_sp.at[idx_d.at[j]], ssem,
                                      add=True)
                     for j in range(_K)]
            for d in descs:
                d.wait()
            return carry

        lax.fori_loop(0, nwin, win, 0)
        plsc.subcore_barrier()
        pltpu.sync_copy(acc_sp.at[pl.ds(s * zr, zr)],
                        out_hbm.at[c, pl.ds(s * zr, zr)])

    return k


def _edge_pass_ones(n_pad, n_rows):
    base = n_rows // _NW
    nwin = base // _K
    assert n_rows == base * _NW and base % _K == 0 and n_pad % (_NS * 8) == 0
    zr = n_pad // _NS
    mesh = plsc.VectorSubcoreMesh(core_axis_name="c", subcore_axis_name="s")

    @functools.partial(
        pl.kernel,
        out_type=jax.ShapeDtypeStruct((_NC, n_pad, 8), jnp.float32),
        mesh=mesh,
        scratch_types=[
            pltpu.VMEM_SHARED((n_pad, 8), jnp.float32),
            pltpu.VMEM((_K, _RW), jnp.int32),
            pltpu.VMEM((_RW, 8), jnp.float32),
            pltpu.SemaphoreType.DMA,
        ],
        compiler_params=pltpu.CompilerParams(use_tc_tiling_on_sc=False),
    )
    def k(dst_hbm, ones_hbm, z_hbm, out_hbm, acc_sp, idx_d, ones_v, ssem):
        c = lax.axis_index("c")
        s = lax.axis_index("s")
        wid = s * _NC + c
        pltpu.sync_copy(z_hbm.at[pl.ds(s * zr, zr)], acc_sp.at[pl.ds(s * zr, zr)])
        pltpu.sync_copy(ones_hbm, ones_v)
        plsc.subcore_barrier()
        start = wid * base

        def win(w, carry):
            r0 = start + w * _K
            pltpu.sync_copy(dst_hbm.at[pl.ds(r0, _K)], idx_d)
            descs = [pltpu.async_copy(ones_v, acc_sp.at[idx_d.at[j]], ssem,
                                      add=True)
                     for j in range(_K)]
            for d in descs:
                d.wait()
            return carry

        lax.fori_loop(0, nwin, win, 0)
        plsc.subcore_barrier()
        pltpu.sync_copy(acc_sp.at[pl.ds(s * zr, zr)],
                        out_hbm.at[c, pl.ds(s * zr, zr)])

    return k


def _dense_a(cnt, xp):
    n = xp.shape[0]
    blk = n // _NS
    grid = n // blk

    def body(cnt_ref, x_ref, y_ref, dinv_ref):
        d = lax.rsqrt(cnt_ref[0][:, 0:1] + cnt_ref[1][:, 0:1] + 1.0)
        dinv_ref[...] = d
        y_ref[...] = x_ref[...] * d

    return pl.pallas_call(
        body,
        grid=(grid,),
        in_specs=[pl.BlockSpec((2, blk, 8), lambda i: (0, i, 0)),
                  pl.BlockSpec((blk, 8), lambda i: (i, 0))],
        out_specs=[pl.BlockSpec((blk, 8), lambda i: (i, 0)),
                   pl.BlockSpec((blk, 1), lambda i: (i, 0))],
        out_shape=[jax.ShapeDtypeStruct((n, 8), jnp.float32),
                   jax.ShapeDtypeStruct((n, 1), jnp.float32)],
    )(cnt, xp)


def _dense_b(acc, y, dinv, w1p, b1r, w2, w3):
    n = y.shape[0]
    blk = n // _NS
    grid = n // blk
    f = w1p.shape[1]

    def body(a_ref, y_ref, d_ref, w1_ref, b1_ref, w2_ref, w3_ref, u_ref):
        d = d_ref[...]
        agg = (a_ref[0] + a_ref[1] + y_ref[...]) * d
        h = jnp.dot(agg, w1_ref[...], preferred_element_type=jnp.float32)
        h = h + b1_ref[...]
        h = h * jax.nn.sigmoid(h)
        z = jnp.dot(jnp.dot(h, w2_ref[...], preferred_element_type=jnp.float32),
                    w3_ref[...], preferred_element_type=jnp.float32)
        u_ref[...] = jnp.broadcast_to(z * d, z.shape[:1] + (8,))

    return pl.pallas_call(
        body,
        grid=(grid,),
        in_specs=[pl.BlockSpec((2, blk, 8), lambda i: (0, i, 0)),
                  pl.BlockSpec((blk, 8), lambda i: (i, 0)),
                  pl.BlockSpec((blk, 1), lambda i: (i, 0)),
                  pl.BlockSpec((8, f), lambda i: (0, 0)),
                  pl.BlockSpec((1, f), lambda i: (0, 0)),
                  pl.BlockSpec(w2.shape, lambda i: (0, 0)),
                  pl.BlockSpec(w3.shape, lambda i: (0, 0))],
        out_specs=pl.BlockSpec((blk, 8), lambda i: (i, 0)),
        out_shape=jax.ShapeDtypeStruct((n, 8), jnp.float32),
    )(acc, y, dinv, w1p, b1r, w2, w3)


def _dense_c(q3, u2, d2, nu2, b2r, w3r, b3r):
    def body(q_ref, u_ref, d_ref, nu_ref, b2_ref, w3_ref, b3_ref, o_ref):
        cval = jnp.sum(b2_ref[...] * w3_ref[...]) + b3_ref[0, 0]
        e = (q_ref[0] + q_ref[1] + u_ref[...]) * d_ref[...] + cval
        o_ref[...] = jnp.sum(e, axis=1, keepdims=True) * nu_ref[...]

    return pl.pallas_call(
        body,
        out_shape=jax.ShapeDtypeStruct((nu2.shape[0], 1), jnp.float32),
    )(q3, u2, d2, nu2, b2r, w3r, b3r)


def _round_up(v, m):
    return (v + m - 1) // m * m


def kernel(x, edge_index, nu, W1, b1, W2, b2, W3, b3):
    n = x.shape[0]
    e = edge_index.shape[1]
    bn = nu.shape[0]
    g = n // bn
    assert n % bn == 0

    n_pad = _round_up(n + 8, _NS * 8)
    n_rows = _round_up(-(-e // _RW), _NW * _K)
    e_pad = n_rows * _RW
    n_sent = n_pad - n
    sent = n + (jnp.arange(e_pad - e, dtype=jnp.int32) % n_sent)
    src2d = jnp.concatenate([edge_index[0], sent]).reshape(n_rows, _RW)
    dst2d = jnp.concatenate([edge_index[1], sent]).reshape(n_rows, _RW)

    xp = jnp.pad(x, ((0, n_pad - n), (0, 8 - x.shape[1])))
    w1p = jnp.pad(W1, ((0, 8 - W1.shape[0]), (0, 0)))
    ones = jnp.ones((_RW, 8), jnp.float32)
    z8 = jnp.zeros((n_pad, 8), jnp.float32)

    cnt = _edge_pass_ones(n_pad, n_rows)(dst2d, ones, z8)
    y, dinv = _dense_a(cnt, xp)
    acc = _edge_pass_gather(n_pad, n_rows, 8)(src2d, dst2d, y, z8)
    u = _dense_b(acc, y, dinv, w1p, b1.reshape(1, -1), W2, W3)
    q = _edge_pass_gather(n_pad, n_rows, 8)(src2d, dst2d, u, z8)
    out = _dense_c(q[:, :n, 0].reshape(_NC, bn, g), u[:n, 0].reshape(bn, g),
                   dinv[:n].reshape(bn, g), nu.reshape(bn, 1),
                   b2.reshape(1, -1), W3.reshape(1, -1), b3.reshape(1, 1))
    return out.reshape(bn)


# speedup vs baseline: 46.4771x; 1.0745x over previous
import functools

import jax
import jax.numpy as jnp
from jax import lax
from jax.experimental import pallas as pl
from jax.experimental.pallas import tpu as pltpu
from jax.experimental.pallas import tpu_sc as plsc

_NC = 2
_NS = 16
_NW = _NC * _NS
_RW = 128
_K = 8


def _edge_pass_gather(n_pad, n_rows, width):
    base = n_rows // _NW
    nwin2 = base // (2 * _K)
    assert n_rows == base * _NW and base % (2 * _K) == 0
    assert n_pad % (_NS * 8) == 0
    zr = n_pad // _NS
    mesh = plsc.VectorSubcoreMesh(core_axis_name="c", subcore_axis_name="s")

    @functools.partial(
        pl.kernel,
        out_type=jax.ShapeDtypeStruct((_NC, n_pad, width), jnp.float32),
        mesh=mesh,
        scratch_types=[
            pltpu.VMEM_SHARED((n_pad, width), jnp.float32),
            pltpu.VMEM((_K, _RW), jnp.int32),
            pltpu.VMEM((_K, _RW), jnp.int32),
            pltpu.VMEM((_K, _RW), jnp.int32),
            pltpu.VMEM((_K, _RW), jnp.int32),
            pltpu.VMEM((_K, _RW, width), jnp.float32),
            pltpu.VMEM((_K, _RW, width), jnp.float32),
            pltpu.SemaphoreType.DMA,
            pltpu.SemaphoreType.DMA,
        ],
        compiler_params=pltpu.CompilerParams(use_tc_tiling_on_sc=False),
    )
    def k(src_hbm, dst_hbm, tab_hbm, z_hbm, out_hbm, acc_sp, idx_sa, idx_da,
          idx_sb, idx_db, gbuf_a, gbuf_b, gsem, ssem):
        c = lax.axis_index("c")
        s = lax.axis_index("s")
        wid = s * _NC + c
        pltpu.sync_copy(z_hbm.at[pl.ds(s * zr, zr)], acc_sp.at[pl.ds(s * zr, zr)])
        plsc.subcore_barrier()
        start = wid * base

        def win(w, carry):
            ra = start + (2 * w) * _K
            rb = ra + _K
            pltpu.sync_copy(src_hbm.at[pl.ds(ra, _K)], idx_sa)
            pltpu.sync_copy(dst_hbm.at[pl.ds(ra, _K)], idx_da)
            ga = [pltpu.async_copy(tab_hbm.at[idx_sa.at[j]], gbuf_a.at[j], gsem)
                  for j in range(_K)]
            pltpu.sync_copy(src_hbm.at[pl.ds(rb, _K)], idx_sb)
            pltpu.sync_copy(dst_hbm.at[pl.ds(rb, _K)], idx_db)
            for d in ga:
                d.wait()
            sa = [pltpu.async_copy(gbuf_a.at[j], acc_sp.at[idx_da.at[j]], ssem,
                                   add=True)
                  for j in range(_K)]
            gb = [pltpu.async_copy(tab_hbm.at[idx_sb.at[j]], gbuf_b.at[j], gsem)
                  for j in range(_K)]
            for d in gb:
                d.wait()
            sb = [pltpu.async_copy(gbuf_b.at[j], acc_sp.at[idx_db.at[j]], ssem,
                                   add=True)
                  for j in range(_K)]
            for d in sa:
                d.wait()
            for d in sb:
                d.wait()
            return carry

        lax.fori_loop(0, nwin2, win, 0)
        plsc.subcore_barrier()
        pltpu.sync_copy(acc_sp.at[pl.ds(s * zr, zr)],
                        out_hbm.at[c, pl.ds(s * zr, zr)])

    return k


def _edge_pass_ones(n_pad, n_rows):
    base = n_rows // _NW
    nwin2 = base // (2 * _K)
    assert n_rows == base * _NW and base % (2 * _K) == 0
    assert n_pad % (_NS * 8) == 0
    zr = n_pad // _NS
    mesh = plsc.VectorSubcoreMesh(core_axis_name="c", subcore_axis_name="s")

    @functools.partial(
        pl.kernel,
        out_type=jax.ShapeDtypeStruct((_NC, n_pad, 8), jnp.float32),
        mesh=mesh,
        scratch_types=[
            pltpu.VMEM_SHARED((n_pad, 8), jnp.float32),
            pltpu.VMEM((_K, _RW), jnp.int32),
            pltpu.VMEM((_K, _RW), jnp.int32),
            pltpu.VMEM((_RW, 8), jnp.float32),
            pltpu.SemaphoreType.DMA,
        ],
        compiler_params=pltpu.CompilerParams(use_tc_tiling_on_sc=False),
    )
    def k(dst_hbm, ones_hbm, z_hbm, out_hbm, acc_sp, idx_da, idx_db, ones_v,
          ssem):
        c = lax.axis_index("c")
        s = lax.axis_index("s")
        wid = s * _NC + c
        pltpu.sync_copy(z_hbm.at[pl.ds(s * zr, zr)], acc_sp.at[pl.ds(s * zr, zr)])
        pltpu.sync_copy(ones_hbm, ones_v)
        plsc.subcore_barrier()
        start = wid * base

        def win(w, carry):
            ra = start + (2 * w) * _K
            rb = ra + _K
            pltpu.sync_copy(dst_hbm.at[pl.ds(ra, _K)], idx_da)
            sa = [pltpu.async_copy(ones_v, acc_sp.at[idx_da.at[j]], ssem,
                                   add=True)
                  for j in range(_K)]
            pltpu.sync_copy(dst_hbm.at[pl.ds(rb, _K)], idx_db)
            sb = [pltpu.async_copy(ones_v, acc_sp.at[idx_db.at[j]], ssem,
                                   add=True)
                  for j in range(_K)]
            for d in sa:
                d.wait()
            for d in sb:
                d.wait()
            return carry

        lax.fori_loop(0, nwin2, win, 0)
        plsc.subcore_barrier()
        pltpu.sync_copy(acc_sp.at[pl.ds(s * zr, zr)],
                        out_hbm.at[c, pl.ds(s * zr, zr)])

    return k


def _dense_a(cnt, xp):
    n = xp.shape[0]
    blk = n // _NS
    grid = n // blk

    def body(cnt_ref, x_ref, y_ref, dinv_ref):
        d = lax.rsqrt(cnt_ref[0][:, 0:1] + cnt_ref[1][:, 0:1] + 1.0)
        dinv_ref[...] = d
        y_ref[...] = x_ref[...] * d

    return pl.pallas_call(
        body,
        grid=(grid,),
        in_specs=[pl.BlockSpec((2, blk, 8), lambda i: (0, i, 0)),
                  pl.BlockSpec((blk, 8), lambda i: (i, 0))],
        out_specs=[pl.BlockSpec((blk, 8), lambda i: (i, 0)),
                   pl.BlockSpec((blk, 1), lambda i: (i, 0))],
        out_shape=[jax.ShapeDtypeStruct((n, 8), jnp.float32),
                   jax.ShapeDtypeStruct((n, 1), jnp.float32)],
    )(cnt, xp)


def _dense_b(acc, y, dinv, w1p, b1r, w2, w3):
    n = y.shape[0]
    blk = n // _NS
    grid = n // blk
    f = w1p.shape[1]

    def body(a_ref, y_ref, d_ref, w1_ref, b1_ref, w2_ref, w3_ref, u_ref):
        d = d_ref[...]
        agg = (a_ref[0] + a_ref[1] + y_ref[...]) * d
        h = jnp.dot(agg, w1_ref[...], preferred_element_type=jnp.float32)
        h = h + b1_ref[...]
        h = h * jax.nn.sigmoid(h)
        z = jnp.dot(jnp.dot(h, w2_ref[...], preferred_element_type=jnp.float32),
                    w3_ref[...], preferred_element_type=jnp.float32)
        u_ref[...] = jnp.broadcast_to(z * d, z.shape[:1] + (8,))

    return pl.pallas_call(
        body,
        grid=(grid,),
        in_specs=[pl.BlockSpec((2, blk, 8), lambda i: (0, i, 0)),
                  pl.BlockSpec((blk, 8), lambda i: (i, 0)),
                  pl.BlockSpec((blk, 1), lambda i: (i, 0)),
                  pl.BlockSpec((8, f), lambda i: (0, 0)),
                  pl.BlockSpec((1, f), lambda i: (0, 0)),
                  pl.BlockSpec(w2.shape, lambda i: (0, 0)),
                  pl.BlockSpec(w3.shape, lambda i: (0, 0))],
        out_specs=pl.BlockSpec((blk, 8), lambda i: (i, 0)),
        out_shape=jax.ShapeDtypeStruct((n, 8), jnp.float32),
    )(acc, y, dinv, w1p, b1r, w2, w3)


def _dense_c(q3, u2, d2, nu2, b2r, w3r, b3r):
    def body(q_ref, u_ref, d_ref, nu_ref, b2_ref, w3_ref, b3_ref, o_ref):
        cval = jnp.sum(b2_ref[...] * w3_ref[...]) + b3_ref[0, 0]
        e = (q_ref[0] + q_ref[1] + u_ref[...]) * d_ref[...] + cval
        o_ref[...] = jnp.sum(e, axis=1, keepdims=True) * nu_ref[...]

    return pl.pallas_call(
        body,
        out_shape=jax.ShapeDtypeStruct((nu2.shape[0], 1), jnp.float32),
    )(q3, u2, d2, nu2, b2r, w3r, b3r)


def _round_up(v, m):
    return (v + m - 1) // m * m


def kernel(x, edge_index, nu, W1, b1, W2, b2, W3, b3):
    n = x.shape[0]
    e = edge_index.shape[1]
    bn = nu.shape[0]
    g = n // bn
    assert n % bn == 0

    n_pad = _round_up(n + 8, _NS * 8)
    n_rows = _round_up(-(-e // _RW), _NW * _K * 2)
    e_pad = n_rows * _RW
    n_sent = n_pad - n
    sent = n + (jnp.arange(e_pad - e, dtype=jnp.int32) % n_sent)
    src2d = jnp.concatenate([edge_index[0], sent]).reshape(n_rows, _RW)
    dst2d = jnp.concatenate([edge_index[1], sent]).reshape(n_rows, _RW)

    xp = jnp.pad(x, ((0, n_pad - n), (0, 8 - x.shape[1])))
    w1p = jnp.pad(W1, ((0, 8 - W1.shape[0]), (0, 0)))
    ones = jnp.ones((_RW, 8), jnp.float32)
    z8 = jnp.zeros((n_pad, 8), jnp.float32)

    cnt = _edge_pass_ones(n_pad, n_rows)(dst2d, ones, z8)
    y, dinv = _dense_a(cnt, xp)
    acc = _edge_pass_gather(n_pad, n_rows, 8)(src2d, dst2d, y, z8)
    u = _dense_b(acc, y, dinv, w1p, b1.reshape(1, -1), W2, W3)
    q = _edge_pass_gather(n_pad, n_rows, 8)(src2d, dst2d, u, z8)
    out = _dense_c(q[:, :n, 0].reshape(_NC, bn, g), u[:n, 0].reshape(bn, g),
                   dinv[:n].reshape(bn, g), nu.reshape(bn, 1),
                   b2.reshape(1, -1), W3.reshape(1, -1), b3.reshape(1, 1))
    return out.reshape(bn)


# speedup vs baseline: 83.9633x; 1.8066x over previous
import functools

import jax
import jax.numpy as jnp
from jax import lax
from jax.experimental import pallas as pl
from jax.experimental.pallas import tpu as pltpu
from jax.experimental.pallas import tpu_sc as plsc

_NC = 2
_NS = 16
_NW = _NC * _NS
_RW = 128
_K = 8


def _edge_pass_gather(n_pad, n_rows, width):
    base = n_rows // _NW
    nwin2 = base // (2 * _K)
    assert n_rows == base * _NW and base % (2 * _K) == 0
    assert n_pad % (_NS * 8) == 0
    zr = n_pad // _NS
    mesh = plsc.VectorSubcoreMesh(core_axis_name="c", subcore_axis_name="s")

    @functools.partial(
        pl.kernel,
        out_type=jax.ShapeDtypeStruct((_NC, n_pad, width), jnp.float32),
        mesh=mesh,
        scratch_types=[
            pltpu.VMEM_SHARED((n_pad, width), jnp.float32),
            pltpu.VMEM((_K, _RW), jnp.int32),
            pltpu.VMEM((_K, _RW), jnp.int32),
            pltpu.VMEM((_K, _RW), jnp.int32),
            pltpu.VMEM((_K, _RW), jnp.int32),
            pltpu.VMEM((_K, _RW, width), jnp.float32),
            pltpu.VMEM((_K, _RW, width), jnp.float32),
            pltpu.SemaphoreType.DMA,
            pltpu.SemaphoreType.DMA,
        ],
        compiler_params=pltpu.CompilerParams(use_tc_tiling_on_sc=False),
    )
    def k(src_hbm, dst_hbm, tab_hbm, z_hbm, out_hbm, acc_sp, idx_sa, idx_da,
          idx_sb, idx_db, gbuf_a, gbuf_b, gsem, ssem):
        c = lax.axis_index("c")
        s = lax.axis_index("s")
        wid = s * _NC + c
        pltpu.sync_copy(z_hbm.at[pl.ds(s * zr, zr)], acc_sp.at[pl.ds(s * zr, zr)])
        plsc.subcore_barrier()
        start = wid * base

        def win(w, carry):
            ra = start + (2 * w) * _K
            rb = ra + _K
            pltpu.sync_copy(src_hbm.at[pl.ds(ra, _K)], idx_sa)
            pltpu.sync_copy(dst_hbm.at[pl.ds(ra, _K)], idx_da)
            ga = [pltpu.async_copy(tab_hbm.at[idx_sa.at[j]], gbuf_a.at[j], gsem)
                  for j in range(_K)]
            pltpu.sync_copy(src_hbm.at[pl.ds(rb, _K)], idx_sb)
            pltpu.sync_copy(dst_hbm.at[pl.ds(rb, _K)], idx_db)
            for d in ga:
                d.wait()
            sa = [pltpu.async_copy(gbuf_a.at[j], acc_sp.at[idx_da.at[j]], ssem,
                                   add=True)
                  for j in range(_K)]
            gb = [pltpu.async_copy(tab_hbm.at[idx_sb.at[j]], gbuf_b.at[j], gsem)
                  for j in range(_K)]
            for d in gb:
                d.wait()
            sb = [pltpu.async_copy(gbuf_b.at[j], acc_sp.at[idx_db.at[j]], ssem,
                                   add=True)
                  for j in range(_K)]
            for d in sa:
                d.wait()
            for d in sb:
                d.wait()
            return carry

        lax.fori_loop(0, nwin2, win, 0)
        plsc.subcore_barrier()
        pltpu.sync_copy(acc_sp.at[pl.ds(s * zr, zr)],
                        out_hbm.at[c, pl.ds(s * zr, zr)])

    return k


def _edge_pass_ones(n_pad, n_rows):
    base = n_rows // _NW
    nwin2 = base // (2 * _K)
    assert n_rows == base * _NW and base % (2 * _K) == 0
    assert n_pad % (_NS * 8) == 0
    zr = n_pad // _NS
    mesh = plsc.VectorSubcoreMesh(core_axis_name="c", subcore_axis_name="s")

    @functools.partial(
        pl.kernel,
        out_type=jax.ShapeDtypeStruct((_NC, n_pad, 8), jnp.float32),
        mesh=mesh,
        scratch_types=[
            pltpu.VMEM_SHARED((n_pad, 8), jnp.float32),
            pltpu.VMEM((_K, _RW), jnp.int32),
            pltpu.VMEM((_K, _RW), jnp.int32),
            pltpu.VMEM((_RW, 8), jnp.float32),
            pltpu.SemaphoreType.DMA,
        ],
        compiler_params=pltpu.CompilerParams(use_tc_tiling_on_sc=False),
    )
    def k(dst_hbm, ones_hbm, z_hbm, out_hbm, acc_sp, idx_da, idx_db, ones_v,
          ssem):
        c = lax.axis_index("c")
        s = lax.axis_index("s")
        wid = s * _NC + c
        pltpu.sync_copy(z_hbm.at[pl.ds(s * zr, zr)], acc_sp.at[pl.ds(s * zr, zr)])
        pltpu.sync_copy(ones_hbm, ones_v)
        plsc.subcore_barrier()
        start = wid * base

        def win(w, carry):
            ra = start + (2 * w) * _K
            rb = ra + _K
            pltpu.sync_copy(dst_hbm.at[pl.ds(ra, _K)], idx_da)
            sa = [pltpu.async_copy(ones_v, acc_sp.at[idx_da.at[j]], ssem,
                                   add=True)
                  for j in range(_K)]
            pltpu.sync_copy(dst_hbm.at[pl.ds(rb, _K)], idx_db)
            sb = [pltpu.async_copy(ones_v, acc_sp.at[idx_db.at[j]], ssem,
                                   add=True)
                  for j in range(_K)]
            for d in sa:
                d.wait()
            for d in sb:
                d.wait()
            return carry

        lax.fori_loop(0, nwin2, win, 0)
        plsc.subcore_barrier()
        pltpu.sync_copy(acc_sp.at[pl.ds(s * zr, zr)],
                        out_hbm.at[c, pl.ds(s * zr, zr)])

    return k


def _dense_a(cnt128, x128):
    r = x128.shape[0]

    def body(cnt_ref, x_ref, y_ref, dinv_ref):
        d = lax.rsqrt(cnt_ref[0] + cnt_ref[1] + 1.0)
        dinv_ref[...] = d
        y_ref[...] = x_ref[...] * d

    return pl.pallas_call(
        body,
        out_shape=[jax.ShapeDtypeStruct((r, 128), jnp.float32),
                   jax.ShapeDtypeStruct((r, 128), jnp.float32)],
    )(cnt128, x128)


def _dense_b(acc128, y128, dinv128, wb, b1t, wz, bc):
    r = y128.shape[0]
    blk = 368
    grid = r // blk
    assert r % blk == 0

    def body(a_ref, y_ref, d_ref, wb_ref, b1_ref, wz_ref, bc_ref, u_ref):
        agg = (a_ref[0] + a_ref[1] + y_ref[...]) * d_ref[...]
        h = jnp.dot(agg, wb_ref[...], preferred_element_type=jnp.float32)
        h = h + b1_ref[...]
        h = h * jax.nn.sigmoid(h)
        z16 = jnp.dot(h, wz_ref[...], preferred_element_type=jnp.float32)
        z128 = jnp.dot(z16, bc_ref[...], preferred_element_type=jnp.float32)
        u_ref[...] = z128 * d_ref[...]

    return pl.pallas_call(
        body,
        grid=(grid,),
        in_specs=[pl.BlockSpec((2, blk, 128), lambda i: (0, i, 0)),
                  pl.BlockSpec((blk, 128), lambda i: (i, 0)),
                  pl.BlockSpec((blk, 128), lambda i: (i, 0)),
                  pl.BlockSpec(wb.shape, lambda i: (0, 0)),
                  pl.BlockSpec(b1t.shape, lambda i: (0, 0)),
                  pl.BlockSpec(wz.shape, lambda i: (0, 0)),
                  pl.BlockSpec(bc.shape, lambda i: (0, 0))],
        out_specs=pl.BlockSpec((blk, 128), lambda i: (i, 0)),
        out_shape=jax.ShapeDtypeStruct((r, 128), jnp.float32),
    )(acc128, y128, dinv128, wb, b1t, wz, bc)


def _dense_c(q0, q1, u3, d3, nu2, b2r, w3r, b3r, g):
    npair, rows, _ = u3.shape
    bound = g * 8

    def body(q0_ref, q1_ref, u_ref, d_ref, nu_ref, b2_ref, w3_ref, b3_ref,
             o_ref):
        cval = jnp.sum(b2_ref[...] * w3_ref[...]) + b3_ref[0, 0]
        e = (q0_ref[...] + q1_ref[...] + u_ref[...]) * d_ref[...] + cval
        row = lax.broadcasted_iota(jnp.int32, e.shape, 1)
        lane = lax.broadcasted_iota(jnp.int32, e.shape, 2)
        in_a = (row * 128 + lane) < bound
        sa = jnp.sum(jnp.where(in_a, e, 0.0), axis=(1, 2))
        sb = jnp.sum(e, axis=(1, 2)) - sa
        o_ref[...] = jnp.stack([sa, sb], axis=1) * nu_ref[...] * 0.125

    return pl.pallas_call(
        body,
        out_shape=jax.ShapeDtypeStruct((npair, 2), jnp.float32),
    )(q0, q1, u3, d3, nu2, b2r, w3r, b3r)


def _round_up(v, m):
    return (v + m - 1) // m * m


def kernel(x, edge_index, nu, W1, b1, W2, b2, W3, b3):
    n = x.shape[0]
    e = edge_index.shape[1]
    bn = nu.shape[0]
    g = n // bn
    assert n % bn == 0 and (2 * g * 8) % 128 == 0 and bn % 2 == 0

    n_pad = _round_up(n + 8, _NS * 8)
    n_rows = _round_up(-(-e // _RW), _NW * _K * 2)
    e_pad = n_rows * _RW
    n_sent = n_pad - n
    r = n_pad // 16
    rn = n // 16
    assert n % 16 == 0 and rn % (bn // 2) == 0
    sent = n + (jnp.arange(e_pad - e, dtype=jnp.int32) % n_sent)
    src2d = jnp.concatenate([edge_index[0], sent]).reshape(n_rows, _RW)
    dst2d = jnp.concatenate([edge_index[1], sent]).reshape(n_rows, _RW)

    x128 = jnp.pad(x, ((0, n_pad - n), (0, 8 - x.shape[1]))).reshape(r, 128)
    w1p = jnp.pad(W1, ((0, 8 - W1.shape[0]), (0, 0)))
    f = w1p.shape[1]
    eye16 = jnp.eye(16, dtype=jnp.float32)
    wb = jnp.kron(eye16, w1p)
    b1t = jnp.tile(b1, 16).reshape(1, 16 * f)
    w23 = W2 @ W3
    wz = jnp.kron(eye16, w23)
    bc = jnp.kron(eye16, jnp.ones((1, 8), jnp.float32))
    ones = jnp.ones((_RW, 8), jnp.float32)
    z8 = jnp.zeros((n_pad, 8), jnp.float32)

    cnt = _edge_pass_ones(n_pad, n_rows)(dst2d, ones, z8)
    y128, dinv128 = _dense_a(cnt.reshape(_NC, r, 128), x128)
    acc = _edge_pass_gather(n_pad, n_rows, 8)(
        src2d, dst2d, y128.reshape(n_pad, 8), z8)
    u128 = _dense_b(acc.reshape(_NC, r, 128), y128, dinv128, wb, b1t, wz, bc)
    q = _edge_pass_gather(n_pad, n_rows, 8)(
        src2d, dst2d, u128.reshape(n_pad, 8), z8)
    rows2 = 2 * g * 8 // 128
    q128 = q.reshape(_NC, r, 128)
    shp = (bn // 2, rows2, 128)
    out = _dense_c(q128[0, :rn].reshape(shp), q128[1, :rn].reshape(shp),
                   u128[:rn].reshape(shp), dinv128[:rn].reshape(shp),
                   nu.reshape(bn // 2, 2), b2.reshape(1, -1),
                   W3.reshape(1, -1), b3.reshape(1, 1), g)
    return out.reshape(bn)


# speedup vs baseline: 90.6469x; 1.0796x over previous
import functools

import jax
import jax.numpy as jnp
from jax import lax
from jax.experimental import pallas as pl
from jax.experimental.pallas import tpu as pltpu
from jax.experimental.pallas import tpu_sc as plsc

_NC = 2
_NS = 16
_NW = _NC * _NS
_RW = 128
_K = 8


def _edge_pass_gather(n_pad, n_rows, width):
    base = n_rows // _NW
    nwin2 = base // (2 * _K)
    assert n_rows == base * _NW and base % (2 * _K) == 0
    assert n_pad % (_NS * 8) == 0
    zr = n_pad // _NS
    mesh = plsc.VectorSubcoreMesh(core_axis_name="c", subcore_axis_name="s")

    @functools.partial(
        pl.kernel,
        out_type=jax.ShapeDtypeStruct((_NC, n_pad, width), jnp.float32),
        mesh=mesh,
        scratch_types=[
            pltpu.VMEM_SHARED((n_pad, width), jnp.float32),
            pltpu.VMEM((_K, _RW), jnp.int32),
            pltpu.VMEM((_K, _RW), jnp.int32),
            pltpu.VMEM((_K, _RW), jnp.int32),
            pltpu.VMEM((_K, _RW), jnp.int32),
            pltpu.VMEM((_K, _RW, width), jnp.float32),
            pltpu.VMEM((_K, _RW, width), jnp.float32),
            pltpu.SemaphoreType.DMA,
            pltpu.SemaphoreType.DMA,
        ],
        compiler_params=pltpu.CompilerParams(use_tc_tiling_on_sc=False),
    )
    def k(src_hbm, dst_hbm, tab_hbm, z_hbm, out_hbm, acc_sp, idx_sa, idx_da,
          idx_sb, idx_db, gbuf_a, gbuf_b, gsem, ssem):
        c = lax.axis_index("c")
        s = lax.axis_index("s")
        wid = s * _NC + c
        pltpu.sync_copy(z_hbm.at[pl.ds(s * zr, zr)], acc_sp.at[pl.ds(s * zr, zr)])
        plsc.subcore_barrier()
        start = wid * base

        def drain(k):
            for _ in range(k):
                pltpu.make_async_copy(z_hbm.at[pl.ds(0, _RW)],
                                      gbuf_a.at[0], ssem).wait()

        def win(w, carry):
            ra = start + (2 * w) * _K
            rb = ra + _K

            @pl.when(w > 0)
            def _():
                drain(_K)
            pltpu.sync_copy(src_hbm.at[pl.ds(ra, _K)], idx_sa)
            pltpu.sync_copy(dst_hbm.at[pl.ds(ra, _K)], idx_da)
            ga = [pltpu.async_copy(tab_hbm.at[idx_sa.at[j]], gbuf_a.at[j], gsem)
                  for j in range(_K)]

            @pl.when(w > 0)
            def _():
                drain(_K)
            pltpu.sync_copy(src_hbm.at[pl.ds(rb, _K)], idx_sb)
            pltpu.sync_copy(dst_hbm.at[pl.ds(rb, _K)], idx_db)
            gb = [pltpu.async_copy(tab_hbm.at[idx_sb.at[j]], gbuf_b.at[j], gsem)
                  for j in range(_K)]
            for d in ga:
                d.wait()
            for j in range(_K):
                pltpu.async_copy(gbuf_a.at[j], acc_sp.at[idx_da.at[j]], ssem,
                                 add=True)
            for d in gb:
                d.wait()
            for j in range(_K):
                pltpu.async_copy(gbuf_b.at[j], acc_sp.at[idx_db.at[j]], ssem,
                                 add=True)
            return carry

        lax.fori_loop(0, nwin2, win, 0)
        drain(2 * _K)
        plsc.subcore_barrier()
        pltpu.sync_copy(acc_sp.at[pl.ds(s * zr, zr)],
                        out_hbm.at[c, pl.ds(s * zr, zr)])

    return k


def _edge_pass_ones(n_pad, n_rows):
    base = n_rows // _NW
    nwin2 = base // (2 * _K)
    assert n_rows == base * _NW and base % (2 * _K) == 0
    assert n_pad % (_NS * 8) == 0
    zr = n_pad // _NS
    mesh = plsc.VectorSubcoreMesh(core_axis_name="c", subcore_axis_name="s")

    @functools.partial(
        pl.kernel,
        out_type=jax.ShapeDtypeStruct((_NC, n_pad, 8), jnp.float32),
        mesh=mesh,
        scratch_types=[
            pltpu.VMEM_SHARED((n_pad, 8), jnp.float32),
            pltpu.VMEM((_K, _RW), jnp.int32),
            pltpu.VMEM((_K, _RW), jnp.int32),
            pltpu.VMEM((_RW, 8), jnp.float32),
            pltpu.SemaphoreType.DMA,
        ],
        compiler_params=pltpu.CompilerParams(use_tc_tiling_on_sc=False),
    )
    def k(dst_hbm, ones_hbm, z_hbm, out_hbm, acc_sp, idx_da, idx_db, ones_v,
          ssem):
        c = lax.axis_index("c")
        s = lax.axis_index("s")
        wid = s * _NC + c
        pltpu.sync_copy(z_hbm.at[pl.ds(s * zr, zr)], acc_sp.at[pl.ds(s * zr, zr)])
        pltpu.sync_copy(ones_hbm, ones_v)
        plsc.subcore_barrier()
        start = wid * base

        def drain(k):
            for _ in range(k):
                pltpu.make_async_copy(z_hbm.at[pl.ds(0, _RW)],
                                      ones_v, ssem).wait()

        def win(w, carry):
            ra = start + (2 * w) * _K
            rb = ra + _K

            @pl.when(w > 0)
            def _():
                drain(_K)
            pltpu.sync_copy(dst_hbm.at[pl.ds(ra, _K)], idx_da)
            for j in range(_K):
                pltpu.async_copy(ones_v, acc_sp.at[idx_da.at[j]], ssem,
                                 add=True)

            @pl.when(w > 0)
            def _():
                drain(_K)
            pltpu.sync_copy(dst_hbm.at[pl.ds(rb, _K)], idx_db)
            for j in range(_K):
                pltpu.async_copy(ones_v, acc_sp.at[idx_db.at[j]], ssem,
                                 add=True)
            return carry

        lax.fori_loop(0, nwin2, win, 0)
        drain(2 * _K)
        plsc.subcore_barrier()
        pltpu.sync_copy(acc_sp.at[pl.ds(s * zr, zr)],
                        out_hbm.at[c, pl.ds(s * zr, zr)])

    return k


def _dense_a(cnt128, x128):
    r = x128.shape[0]

    def body(cnt_ref, x_ref, y_ref, dinv_ref):
        d = lax.rsqrt(cnt_ref[0] + cnt_ref[1] + 1.0)
        dinv_ref[...] = d
        y_ref[...] = x_ref[...] * d

    return pl.pallas_call(
        body,
        out_shape=[jax.ShapeDtypeStruct((r, 128), jnp.float32),
                   jax.ShapeDtypeStruct((r, 128), jnp.float32)],
    )(cnt128, x128)


def _dense_b(acc128, y128, dinv128, wb, b1t, wz, bc):
    r = y128.shape[0]
    blk = 368
    grid = r // blk
    assert r % blk == 0

    def body(a_ref, y_ref, d_ref, wb_ref, b1_ref, wz_ref, bc_ref, u_ref):
        agg = (a_ref[0] + a_ref[1] + y_ref[...]) * d_ref[...]
        h = jnp.dot(agg, wb_ref[...], preferred_element_type=jnp.float32)
        h = h + b1_ref[...]
        h = h * jax.nn.sigmoid(h)
        z16 = jnp.dot(h, wz_ref[...], preferred_element_type=jnp.float32)
        z128 = jnp.dot(z16, bc_ref[...], preferred_element_type=jnp.float32)
        u_ref[...] = z128 * d_ref[...]

    return pl.pallas_call(
        body,
        grid=(grid,),
        in_specs=[pl.BlockSpec((2, blk, 128), lambda i: (0, i, 0)),
                  pl.BlockSpec((blk, 128), lambda i: (i, 0)),
                  pl.BlockSpec((blk, 128), lambda i: (i, 0)),
                  pl.BlockSpec(wb.shape, lambda i: (0, 0)),
                  pl.BlockSpec(b1t.shape, lambda i: (0, 0)),
                  pl.BlockSpec(wz.shape, lambda i: (0, 0)),
                  pl.BlockSpec(bc.shape, lambda i: (0, 0))],
        out_specs=pl.BlockSpec((blk, 128), lambda i: (i, 0)),
        out_shape=jax.ShapeDtypeStruct((r, 128), jnp.float32),
    )(acc128, y128, dinv128, wb, b1t, wz, bc)


def _dense_c(q0, q1, u3, d3, nu2, b2r, w3r, b3r, g):
    npair, rows, _ = u3.shape
    bound = g * 8

    def body(q0_ref, q1_ref, u_ref, d_ref, nu_ref, b2_ref, w3_ref, b3_ref,
             o_ref):
        cval = jnp.sum(b2_ref[...] * w3_ref[...]) + b3_ref[0, 0]
        e = (q0_ref[...] + q1_ref[...] + u_ref[...]) * d_ref[...] + cval
        row = lax.broadcasted_iota(jnp.int32, e.shape, 1)
        lane = lax.broadcasted_iota(jnp.int32, e.shape, 2)
        in_a = (row * 128 + lane) < bound
        sa = jnp.sum(jnp.where(in_a, e, 0.0), axis=(1, 2))
        sb = jnp.sum(e, axis=(1, 2)) - sa
        o_ref[...] = jnp.stack([sa, sb], axis=1) * nu_ref[...] * 0.125

    return pl.pallas_call(
        body,
        out_shape=jax.ShapeDtypeStruct((npair, 2), jnp.float32),
    )(q0, q1, u3, d3, nu2, b2r, w3r, b3r)


def _round_up(v, m):
    return (v + m - 1) // m * m


def kernel(x, edge_index, nu, W1, b1, W2, b2, W3, b3):
    n = x.shape[0]
    e = edge_index.shape[1]
    bn = nu.shape[0]
    g = n // bn
    assert n % bn == 0 and (2 * g * 8) % 128 == 0 and bn % 2 == 0

    n_pad = _round_up(n + 8, _NS * 8)
    n_rows = _round_up(-(-e // _RW), _NW * _K * 2)
    e_pad = n_rows * _RW
    n_sent = n_pad - n
    r = n_pad // 16
    rn = n // 16
    assert n % 16 == 0 and rn % (bn // 2) == 0
    sent = n + (jnp.arange(e_pad - e, dtype=jnp.int32) % n_sent)
    dst2d = jnp.concatenate([edge_index[1], sent]).reshape(n_rows, _RW)
    dst2d = lax.optimization_barrier(dst2d)
    src2d = jnp.concatenate([edge_index[0], sent]).reshape(n_rows, _RW)

    x128 = jnp.pad(x, ((0, n_pad - n), (0, 8 - x.shape[1]))).reshape(r, 128)
    w1p = jnp.pad(W1, ((0, 8 - W1.shape[0]), (0, 0)))
    f = w1p.shape[1]
    eye16 = jnp.eye(16, dtype=jnp.float32)
    wb = jnp.kron(eye16, w1p)
    b1t = jnp.tile(b1, 16).reshape(1, 16 * f)
    w23 = W2 @ W3
    wz = jnp.kron(eye16, w23)
    bc = jnp.kron(eye16, jnp.ones((1, 8), jnp.float32))
    ones = jnp.ones((_RW, 8), jnp.float32)
    z8 = jnp.zeros((n_pad, 8), jnp.float32)

    cnt = _edge_pass_ones(n_pad, n_rows)(dst2d, ones, z8)
    y128, dinv128 = _dense_a(cnt.reshape(_NC, r, 128), x128)
    acc = _edge_pass_gather(n_pad, n_rows, 8)(
        src2d, dst2d, y128.reshape(n_pad, 8), z8)
    u128 = _dense_b(acc.reshape(_NC, r, 128), y128, dinv128, wb, b1t, wz, bc)
    q = _edge_pass_gather(n_pad, n_rows, 8)(
        src2d, dst2d, u128.reshape(n_pad, 8), z8)
    rows2 = 2 * g * 8 // 128
    q128 = q.reshape(_NC, r, 128)
    shp = (bn // 2, rows2, 128)
    out = _dense_c(q128[0, :rn].reshape(shp), q128[1, :rn].reshape(shp),
                   u128[:rn].reshape(shp), dinv128[:rn].reshape(shp),
                   nu.reshape(bn // 2, 2), b2.reshape(1, -1),
                   W3.reshape(1, -1), b3.reshape(1, 1), g)
    return out.reshape(bn)


# speedup vs baseline: 104.4943x; 1.1528x over previous
import functools

import jax
import jax.numpy as jnp
from jax import lax
from jax.experimental import pallas as pl
from jax.experimental.pallas import tpu as pltpu
from jax.experimental.pallas import tpu_sc as plsc

_NC = 2
_NS = 16
_NW = _NC * _NS
_RW = 128
_K = 8


def _edge_pass_gather(n_pad, n_rows, width):
    base = n_rows // _NW
    nwin2 = base // (2 * _K)
    assert n_rows == base * _NW and base % (2 * _K) == 0
    assert n_pad % (_NS * 8) == 0
    zr = n_pad // _NS
    mesh = plsc.VectorSubcoreMesh(core_axis_name="c", subcore_axis_name="s")

    @functools.partial(
        pl.kernel,
        out_type=jax.ShapeDtypeStruct((_NC, n_pad, width), jnp.float32),
        mesh=mesh,
        scratch_types=[
            pltpu.VMEM_SHARED((n_pad, width), jnp.float32),
            pltpu.VMEM_SHARED((n_pad, width), jnp.float32),
            pltpu.VMEM((_K, _RW), jnp.int32),
            pltpu.VMEM((_K, _RW), jnp.int32),
            pltpu.VMEM((_K, _RW), jnp.int32),
            pltpu.VMEM((_K, _RW), jnp.int32),
            pltpu.VMEM((_K, _RW, width), jnp.float32),
            pltpu.VMEM((_K, _RW, width), jnp.float32),
            pltpu.SemaphoreType.DMA,
            pltpu.SemaphoreType.DMA,
        ],
        compiler_params=pltpu.CompilerParams(use_tc_tiling_on_sc=False),
    )
    def k(src_hbm, dst_hbm, tab_hbm, z_hbm, out_hbm, acc_sp, tab_sp, idx_sa,
          idx_da, idx_sb, idx_db, gbuf_a, gbuf_b, gsem, ssem):
        c = lax.axis_index("c")
        s = lax.axis_index("s")
        wid = s * _NC + c
        pltpu.sync_copy(z_hbm.at[pl.ds(s * zr, zr)], acc_sp.at[pl.ds(s * zr, zr)])
        pltpu.sync_copy(tab_hbm.at[pl.ds(s * zr, zr)], tab_sp.at[pl.ds(s * zr, zr)])
        plsc.subcore_barrier()
        start = wid * base

        def drain(k):
            for _ in range(k):
                pltpu.make_async_copy(z_hbm.at[pl.ds(0, _RW)],
                                      gbuf_a.at[0], ssem).wait()

        def win(w, carry):
            ra = start + (2 * w) * _K
            rb = ra + _K

            @pl.when(w > 0)
            def _():
                drain(_K)
            pltpu.sync_copy(src_hbm.at[pl.ds(ra, _K)], idx_sa)
            pltpu.sync_copy(dst_hbm.at[pl.ds(ra, _K)], idx_da)
            ga = [pltpu.async_copy(tab_sp.at[idx_sa.at[j]], gbuf_a.at[j], gsem)
                  for j in range(_K)]

            @pl.when(w > 0)
            def _():
                drain(_K)
            pltpu.sync_copy(src_hbm.at[pl.ds(rb, _K)], idx_sb)
            pltpu.sync_copy(dst_hbm.at[pl.ds(rb, _K)], idx_db)
            gb = [pltpu.async_copy(tab_sp.at[idx_sb.at[j]], gbuf_b.at[j], gsem)
                  for j in range(_K)]
            for d in ga:
                d.wait()
            for j in range(_K):
                pltpu.async_copy(gbuf_a.at[j], acc_sp.at[idx_da.at[j]], ssem,
                                 add=True)
            for d in gb:
                d.wait()
            for j in range(_K):
                pltpu.async_copy(gbuf_b.at[j], acc_sp.at[idx_db.at[j]], ssem,
                                 add=True)
            return carry

        lax.fori_loop(0, nwin2, win, 0)
        drain(2 * _K)
        plsc.subcore_barrier()
        pltpu.sync_copy(acc_sp.at[pl.ds(s * zr, zr)],
                        out_hbm.at[c, pl.ds(s * zr, zr)])

    return k


def _edge_pass_ones(n_pad, n_rows):
    base = n_rows // _NW
    nwin2 = base // (2 * _K)
    assert n_rows == base * _NW and base % (2 * _K) == 0
    assert n_pad % (_NS * 8) == 0
    zr = n_pad // _NS
    mesh = plsc.VectorSubcoreMesh(core_axis_name="c", subcore_axis_name="s")

    @functools.partial(
        pl.kernel,
        out_type=jax.ShapeDtypeStruct((_NC, n_pad, 8), jnp.float32),
        mesh=mesh,
        scratch_types=[
            pltpu.VMEM_SHARED((n_pad, 8), jnp.float32),
            pltpu.VMEM((_K, _RW), jnp.int32),
            pltpu.VMEM((_K, _RW), jnp.int32),
            pltpu.VMEM((_RW, 8), jnp.float32),
            pltpu.SemaphoreType.DMA,
        ],
        compiler_params=pltpu.CompilerParams(use_tc_tiling_on_sc=False),
    )
    def k(dst_hbm, ones_hbm, z_hbm, out_hbm, acc_sp, idx_da, idx_db, ones_v,
          ssem):
        c = lax.axis_index("c")
        s = lax.axis_index("s")
        wid = s * _NC + c
        pltpu.sync_copy(z_hbm.at[pl.ds(s * zr, zr)], acc_sp.at[pl.ds(s * zr, zr)])
        pltpu.sync_copy(ones_hbm, ones_v)
        plsc.subcore_barrier()
        start = wid * base

        def drain(k):
            for _ in range(k):
                pltpu.make_async_copy(z_hbm.at[pl.ds(0, _RW)],
                                      ones_v, ssem).wait()

        def win(w, carry):
            ra = start + (2 * w) * _K
            rb = ra + _K

            @pl.when(w > 0)
            def _():
                drain(_K)
            pltpu.sync_copy(dst_hbm.at[pl.ds(ra, _K)], idx_da)
            for j in range(_K):
                pltpu.async_copy(ones_v, acc_sp.at[idx_da.at[j]], ssem,
                                 add=True)

            @pl.when(w > 0)
            def _():
                drain(_K)
            pltpu.sync_copy(dst_hbm.at[pl.ds(rb, _K)], idx_db)
            for j in range(_K):
                pltpu.async_copy(ones_v, acc_sp.at[idx_db.at[j]], ssem,
                                 add=True)
            return carry

        lax.fori_loop(0, nwin2, win, 0)
        drain(2 * _K)
        plsc.subcore_barrier()
        pltpu.sync_copy(acc_sp.at[pl.ds(s * zr, zr)],
                        out_hbm.at[c, pl.ds(s * zr, zr)])

    return k


def _dense_a(cnt128, x128):
    r = x128.shape[0]

    def body(cnt_ref, x_ref, y_ref, dinv_ref):
        d = lax.rsqrt(cnt_ref[0] + cnt_ref[1] + 1.0)
        dinv_ref[...] = d
        y_ref[...] = x_ref[...] * d

    return pl.pallas_call(
        body,
        out_shape=[jax.ShapeDtypeStruct((r, 128), jnp.float32),
                   jax.ShapeDtypeStruct((r, 128), jnp.float32)],
    )(cnt128, x128)


def _dense_b(acc128, y128, dinv128, wb, b1t, wz, bc):
    r = y128.shape[0]
    blk = 368
    grid = r // blk
    assert r % blk == 0

    def body(a_ref, y_ref, d_ref, wb_ref, b1_ref, wz_ref, bc_ref, u_ref):
        agg = (a_ref[0] + a_ref[1] + y_ref[...]) * d_ref[...]
        h = jnp.dot(agg, wb_ref[...], preferred_element_type=jnp.float32)
        h = h + b1_ref[...]
        h = h * jax.nn.sigmoid(h)
        z16 = jnp.dot(h, wz_ref[...], preferred_element_type=jnp.float32)
        z128 = jnp.dot(z16, bc_ref[...], preferred_element_type=jnp.float32)
        u_ref[...] = z128 * d_ref[...]

    return pl.pallas_call(
        body,
        grid=(grid,),
        in_specs=[pl.BlockSpec((2, blk, 128), lambda i: (0, i, 0)),
                  pl.BlockSpec((blk, 128), lambda i: (i, 0)),
                  pl.BlockSpec((blk, 128), lambda i: (i, 0)),
                  pl.BlockSpec(wb.shape, lambda i: (0, 0)),
                  pl.BlockSpec(b1t.shape, lambda i: (0, 0)),
                  pl.BlockSpec(wz.shape, lambda i: (0, 0)),
                  pl.BlockSpec(bc.shape, lambda i: (0, 0))],
        out_specs=pl.BlockSpec((blk, 128), lambda i: (i, 0)),
        out_shape=jax.ShapeDtypeStruct((r, 128), jnp.float32),
    )(acc128, y128, dinv128, wb, b1t, wz, bc)


def _dense_c(q0, q1, u3, d3, nu2, b2r, w3r, b3r, g):
    npair, rows, _ = u3.shape
    bound = g * 8

    def body(q0_ref, q1_ref, u_ref, d_ref, nu_ref, b2_ref, w3_ref, b3_ref,
             o_ref):
        cval = jnp.sum(b2_ref[...] * w3_ref[...]) + b3_ref[0, 0]
        e = (q0_ref[...] + q1_ref[...] + u_ref[...]) * d_ref[...] + cval
        row = lax.broadcasted_iota(jnp.int32, e.shape, 1)
        lane = lax.broadcasted_iota(jnp.int32, e.shape, 2)
        in_a = (row * 128 + lane) < bound
        sa = jnp.sum(jnp.where(in_a, e, 0.0), axis=(1, 2))
        sb = jnp.sum(e, axis=(1, 2)) - sa
        o_ref[...] = jnp.stack([sa, sb], axis=1) * nu_ref[...] * 0.125

    return pl.pallas_call(
        body,
        out_shape=jax.ShapeDtypeStruct((npair, 2), jnp.float32),
    )(q0, q1, u3, d3, nu2, b2r, w3r, b3r)


def _round_up(v, m):
    return (v + m - 1) // m * m


def kernel(x, edge_index, nu, W1, b1, W2, b2, W3, b3):
    n = x.shape[0]
    e = edge_index.shape[1]
    bn = nu.shape[0]
    g = n // bn
    assert n % bn == 0 and (2 * g * 8) % 128 == 0 and bn % 2 == 0

    n_pad = _round_up(n + 8, _NS * 8)
    n_rows = _round_up(-(-e // _RW), _NW * _K * 2)
    e_pad = n_rows * _RW
    n_sent = n_pad - n
    r = n_pad // 16
    rn = n // 16
    assert n % 16 == 0 and rn % (bn // 2) == 0
    sent = n + (jnp.arange(e_pad - e, dtype=jnp.int32) % n_sent)
    dst2d = jnp.concatenate([edge_index[1], sent]).reshape(n_rows, _RW)
    dst2d = lax.optimization_barrier(dst2d)
    src2d = jnp.concatenate([edge_index[0], sent]).reshape(n_rows, _RW)

    x128 = jnp.pad(x, ((0, n_pad - n), (0, 8 - x.shape[1]))).reshape(r, 128)
    w1p = jnp.pad(W1, ((0, 8 - W1.shape[0]), (0, 0)))
    f = w1p.shape[1]
    eye16 = jnp.eye(16, dtype=jnp.float32)
    wb = jnp.kron(eye16, w1p)
    b1t = jnp.tile(b1, 16).reshape(1, 16 * f)
    w23 = W2 @ W3
    wz = jnp.kron(eye16, w23)
    bc = jnp.kron(eye16, jnp.ones((1, 8), jnp.float32))
    ones = jnp.ones((_RW, 8), jnp.float32)
    z8 = jnp.zeros((n_pad, 8), jnp.float32)

    cnt = _edge_pass_ones(n_pad, n_rows)(dst2d, ones, z8)
    y128, dinv128 = _dense_a(cnt.reshape(_NC, r, 128), x128)
    acc = _edge_pass_gather(n_pad, n_rows, 8)(
        src2d, dst2d, y128.reshape(n_pad, 8), z8)
    u128 = _dense_b(acc.reshape(_NC, r, 128), y128, dinv128, wb, b1t, wz, bc)
    q = _edge_pass_gather(n_pad, n_rows, 8)(
        src2d, dst2d, u128.reshape(n_pad, 8), z8)
    rows2 = 2 * g * 8 // 128
    q128 = q.reshape(_NC, r, 128)
    shp = (bn // 2, rows2, 128)
    out = _dense_c(q128[0, :rn].reshape(shp), q128[1, :rn].reshape(shp),
                   u128[:rn].reshape(shp), dinv128[:rn].reshape(shp),
                   nu.reshape(bn // 2, 2), b2.reshape(1, -1),
                   W3.reshape(1, -1), b3.reshape(1, 1), g)
    return out.reshape(bn)


# speedup vs baseline: 114.6104x; 1.0968x over previous
import functools

import jax
import jax.numpy as jnp
from jax import lax
from jax.experimental import pallas as pl
from jax.experimental.pallas import tpu as pltpu
from jax.experimental.pallas import tpu_sc as plsc

_NC = 2
_NS = 16
_NW = _NC * _NS
_RW = 128
_K = 8


def _edge_pass_gather(n_pad, n_rows, width):
    base = n_rows // _NW
    nwin2 = base // (2 * _K)
    assert n_rows == base * _NW and base % (2 * _K) == 0
    assert n_pad % (_NS * 8) == 0
    zr = n_pad // _NS
    mesh = plsc.VectorSubcoreMesh(core_axis_name="c", subcore_axis_name="s")

    @functools.partial(
        pl.kernel,
        out_type=jax.ShapeDtypeStruct((_NC, n_pad, width), jnp.float32),
        mesh=mesh,
        scratch_types=[
            pltpu.VMEM_SHARED((n_pad, width), jnp.float32),
            pltpu.VMEM_SHARED((n_pad, width), jnp.float32),
            pltpu.VMEM((_K, _RW), jnp.int32),
            pltpu.VMEM((_K, _RW), jnp.int32),
            pltpu.VMEM((_K, _RW), jnp.int32),
            pltpu.VMEM((_K, _RW), jnp.int32),
            pltpu.VMEM((_K, _RW, width), jnp.float32),
            pltpu.VMEM((_K, _RW, width), jnp.float32),
            pltpu.SemaphoreType.DMA,
            pltpu.SemaphoreType.DMA,
            pltpu.SemaphoreType.DMA,
        ],
        compiler_params=pltpu.CompilerParams(use_tc_tiling_on_sc=False),
    )
    def k(src_hbm, dst_hbm, tab_hbm, z_hbm, out_hbm, acc_sp, tab_sp, idx_sa,
          idx_da, idx_sb, idx_db, gbuf_a, gbuf_b, gsem, ssem, lsem):
        c = lax.axis_index("c")
        s = lax.axis_index("s")
        wid = s * _NC + c
        pltpu.sync_copy(z_hbm.at[pl.ds(s * zr, zr)], acc_sp.at[pl.ds(s * zr, zr)])
        pltpu.sync_copy(tab_hbm.at[pl.ds(s * zr, zr)], tab_sp.at[pl.ds(s * zr, zr)])
        plsc.subcore_barrier()
        start = wid * base

        def drain(k):
            for _ in range(k):
                pltpu.make_async_copy(z_hbm.at[pl.ds(0, _RW)],
                                      gbuf_a.at[0], ssem).wait()

        def win(w, carry):
            ra = start + (2 * w) * _K
            rb = ra + _K

            @pl.when(w > 0)
            def _():
                drain(2 * _K)
            loads = [pltpu.async_copy(src_hbm.at[pl.ds(ra, _K)], idx_sa, lsem),
                     pltpu.async_copy(dst_hbm.at[pl.ds(ra, _K)], idx_da, lsem),
                     pltpu.async_copy(src_hbm.at[pl.ds(rb, _K)], idx_sb, lsem),
                     pltpu.async_copy(dst_hbm.at[pl.ds(rb, _K)], idx_db, lsem)]
            for d in loads:
                d.wait()
            ga = [pltpu.async_copy(tab_sp.at[idx_sa.at[j]], gbuf_a.at[j], gsem)
                  for j in range(_K)]
            gb = [pltpu.async_copy(tab_sp.at[idx_sb.at[j]], gbuf_b.at[j], gsem)
                  for j in range(_K)]
            for d in ga:
                d.wait()
            for j in range(_K):
                pltpu.async_copy(gbuf_a.at[j], acc_sp.at[idx_da.at[j]], ssem,
                                 add=True)
            for d in gb:
                d.wait()
            for j in range(_K):
                pltpu.async_copy(gbuf_b.at[j], acc_sp.at[idx_db.at[j]], ssem,
                                 add=True)
            return carry

        lax.fori_loop(0, nwin2, win, 0)
        drain(2 * _K)
        plsc.subcore_barrier()
        pltpu.sync_copy(acc_sp.at[pl.ds(s * zr, zr)],
                        out_hbm.at[c, pl.ds(s * zr, zr)])

    return k


def _edge_pass_ones(n_pad, n_rows):
    base = n_rows // _NW
    nwin2 = base // (2 * _K)
    assert n_rows == base * _NW and base % (2 * _K) == 0
    assert n_pad % (_NS * 8) == 0
    zr = n_pad // _NS
    mesh = plsc.VectorSubcoreMesh(core_axis_name="c", subcore_axis_name="s")

    @functools.partial(
        pl.kernel,
        out_type=jax.ShapeDtypeStruct((_NC, n_pad, 8), jnp.float32),
        mesh=mesh,
        scratch_types=[
            pltpu.VMEM_SHARED((n_pad, 8), jnp.float32),
            pltpu.VMEM((_K, _RW), jnp.int32),
            pltpu.VMEM((_K, _RW), jnp.int32),
            pltpu.VMEM((_RW, 8), jnp.float32),
            pltpu.SemaphoreType.DMA,
        ],
        compiler_params=pltpu.CompilerParams(use_tc_tiling_on_sc=False),
    )
    def k(dst_hbm, ones_hbm, z_hbm, out_hbm, acc_sp, idx_da, idx_db, ones_v,
          ssem):
        c = lax.axis_index("c")
        s = lax.axis_index("s")
        wid = s * _NC + c
        pltpu.sync_copy(z_hbm.at[pl.ds(s * zr, zr)], acc_sp.at[pl.ds(s * zr, zr)])
        pltpu.sync_copy(ones_hbm, ones_v)
        plsc.subcore_barrier()
        start = wid * base

        def drain(k):
            for _ in range(k):
                pltpu.make_async_copy(z_hbm.at[pl.ds(0, _RW)],
                                      ones_v, ssem).wait()

        def win(w, carry):
            ra = start + (2 * w) * _K
            rb = ra + _K

            @pl.when(w > 0)
            def _():
                drain(_K)
            pltpu.sync_copy(dst_hbm.at[pl.ds(ra, _K)], idx_da)
            for j in range(_K):
                pltpu.async_copy(ones_v, acc_sp.at[idx_da.at[j]], ssem,
                                 add=True)

            @pl.when(w > 0)
            def _():
                drain(_K)
            pltpu.sync_copy(dst_hbm.at[pl.ds(rb, _K)], idx_db)
            for j in range(_K):
                pltpu.async_copy(ones_v, acc_sp.at[idx_db.at[j]], ssem,
                                 add=True)
            return carry

        lax.fori_loop(0, nwin2, win, 0)
        drain(2 * _K)
        plsc.subcore_barrier()
        pltpu.sync_copy(acc_sp.at[pl.ds(s * zr, zr)],
                        out_hbm.at[c, pl.ds(s * zr, zr)])

    return k


def _dense_a(cnt128, x128):
    r = x128.shape[0]

    def body(cnt_ref, x_ref, y_ref, dinv_ref):
        d = lax.rsqrt(cnt_ref[0] + cnt_ref[1] + 1.0)
        dinv_ref[...] = d
        y_ref[...] = x_ref[...] * d

    return pl.pallas_call(
        body,
        out_shape=[jax.ShapeDtypeStruct((r, 128), jnp.float32),
                   jax.ShapeDtypeStruct((r, 128), jnp.float32)],
    )(cnt128, x128)


def _dense_b(acc128, y128, dinv128, wb, b1t, wz, bc):
    r = y128.shape[0]
    blk = 368
    grid = r // blk
    assert r % blk == 0

    def body(a_ref, y_ref, d_ref, wb_ref, b1_ref, wz_ref, bc_ref, u_ref):
        agg = (a_ref[0] + a_ref[1] + y_ref[...]) * d_ref[...]
        h = jnp.dot(agg, wb_ref[...], preferred_element_type=jnp.float32)
        h = h + b1_ref[...]
        h = h * jax.nn.sigmoid(h)
        z16 = jnp.dot(h, wz_ref[...], preferred_element_type=jnp.float32)
        z128 = jnp.dot(z16, bc_ref[...], preferred_element_type=jnp.float32)
        u_ref[...] = z128 * d_ref[...]

    return pl.pallas_call(
        body,
        grid=(grid,),
        in_specs=[pl.BlockSpec((2, blk, 128), lambda i: (0, i, 0)),
                  pl.BlockSpec((blk, 128), lambda i: (i, 0)),
                  pl.BlockSpec((blk, 128), lambda i: (i, 0)),
                  pl.BlockSpec(wb.shape, lambda i: (0, 0)),
                  pl.BlockSpec(b1t.shape, lambda i: (0, 0)),
                  pl.BlockSpec(wz.shape, lambda i: (0, 0)),
                  pl.BlockSpec(bc.shape, lambda i: (0, 0))],
        out_specs=pl.BlockSpec((blk, 128), lambda i: (i, 0)),
        out_shape=jax.ShapeDtypeStruct((r, 128), jnp.float32),
    )(acc128, y128, dinv128, wb, b1t, wz, bc)


def _dense_c(q0, q1, u3, d3, nu2, b2r, w3r, b3r, g):
    npair, rows, _ = u3.shape
    bound = g * 8

    def body(q0_ref, q1_ref, u_ref, d_ref, nu_ref, b2_ref, w3_ref, b3_ref,
             o_ref):
        cval = jnp.sum(b2_ref[...] * w3_ref[...]) + b3_ref[0, 0]
        e = (q0_ref[...] + q1_ref[...] + u_ref[...]) * d_ref[...] + cval
        row = lax.broadcasted_iota(jnp.int32, e.shape, 1)
        lane = lax.broadcasted_iota(jnp.int32, e.shape, 2)
        in_a = (row * 128 + lane) < bound
        sa = jnp.sum(jnp.where(in_a, e, 0.0), axis=(1, 2))
        sb = jnp.sum(e, axis=(1, 2)) - sa
        o_ref[...] = jnp.stack([sa, sb], axis=1) * nu_ref[...] * 0.125

    return pl.pallas_call(
        body,
        out_shape=jax.ShapeDtypeStruct((npair, 2), jnp.float32),
    )(q0, q1, u3, d3, nu2, b2r, w3r, b3r)


def _round_up(v, m):
    return (v + m - 1) // m * m


def kernel(x, edge_index, nu, W1, b1, W2, b2, W3, b3):
    n = x.shape[0]
    e = edge_index.shape[1]
    bn = nu.shape[0]
    g = n // bn
    assert n % bn == 0 and (2 * g * 8) % 128 == 0 and bn % 2 == 0

    n_pad = _round_up(n + 8, _NS * 8)
    n_rows = _round_up(-(-e // _RW), _NW * _K * 2)
    e_pad = n_rows * _RW
    n_sent = n_pad - n
    r = n_pad // 16
    rn = n // 16
    assert n % 16 == 0 and rn % (bn // 2) == 0
    sent = n + (jnp.arange(e_pad - e, dtype=jnp.int32) % n_sent)
    dst2d = jnp.concatenate([edge_index[1], sent]).reshape(n_rows, _RW)
    dst2d = lax.optimization_barrier(dst2d)
    src2d = jnp.concatenate([edge_index[0], sent]).reshape(n_rows, _RW)

    x128 = jnp.pad(x, ((0, n_pad - n), (0, 8 - x.shape[1]))).reshape(r, 128)
    w1p = jnp.pad(W1, ((0, 8 - W1.shape[0]), (0, 0)))
    f = w1p.shape[1]
    eye16 = jnp.eye(16, dtype=jnp.float32)
    wb = jnp.kron(eye16, w1p)
    b1t = jnp.tile(b1, 16).reshape(1, 16 * f)
    w23 = W2 @ W3
    wz = jnp.kron(eye16, w23)
    bc = jnp.kron(eye16, jnp.ones((1, 8), jnp.float32))
    ones = jnp.ones((_RW, 8), jnp.float32)
    z8 = jnp.zeros((n_pad, 8), jnp.float32)

    cnt = _edge_pass_ones(n_pad, n_rows)(dst2d, ones, z8)
    y128, dinv128 = _dense_a(cnt.reshape(_NC, r, 128), x128)
    acc = _edge_pass_gather(n_pad, n_rows, 8)(
        src2d, dst2d, y128.reshape(n_pad, 8), z8)
    u128 = _dense_b(acc.reshape(_NC, r, 128), y128, dinv128, wb, b1t, wz, bc)
    q = _edge_pass_gather(n_pad, n_rows, 8)(
        src2d, dst2d, u128.reshape(n_pad, 8), z8)
    rows2 = 2 * g * 8 // 128
    q128 = q.reshape(_NC, r, 128)
    shp = (bn // 2, rows2, 128)
    out = _dense_c(q128[0, :rn].reshape(shp), q128[1, :rn].reshape(shp),
                   u128[:rn].reshape(shp), dinv128[:rn].reshape(shp),
                   nu.reshape(bn // 2, 2), b2.reshape(1, -1),
                   W3.reshape(1, -1), b3.reshape(1, 1), g)
    return out.reshape(bn)
